# Initial kernel scaffold; baseline (speedup 1.0000x reference)
import jax
import jax.numpy as jnp
from jax.experimental import pallas as pl


def kernel(input_ids, table, fc_w, fc_b):
    raise NotImplementedError("write your pallas kernel here")


# speedup vs baseline: 115.9655x; 115.9655x over previous
import functools

import jax
import jax.numpy as jnp
from jax import lax
from jax.experimental import pallas as pl
from jax.experimental.pallas import tpu as pltpu
from jax.experimental.pallas import tpu_sc as plsc

_VOCAB_PAD = 1008
_EMB = 8
_OUT = 4
_LANES = 16
_NC = 2
_NS = 16
_NW = _NC * _NS


def _project_table(table_pad, fc_w, inv_l):
    def body(t_ref, w_ref, p_ref):
        t = t_ref[...]
        row = lax.broadcasted_iota(jnp.int32, t.shape, 0)
        t = jnp.where(row == 0, 0.0, t)
        p = lax.dot_general(t, w_ref[...], (((1,), (1,)), ((), ())),
                            preferred_element_type=jnp.float32)
        p_ref[...] = p * inv_l

    return pl.pallas_call(
        body,
        out_shape=jax.ShapeDtypeStruct((_VOCAB_PAD, _OUT), jnp.float32),
    )(table_pad, fc_w)


def _make_sc_kernel(B, L):
    rows_per_tile = B // _NW
    chunk = 64
    n_chunks = rows_per_tile // chunk
    groups = chunk // _LANES

    mesh = plsc.VectorSubcoreMesh(core_axis_name="c", subcore_axis_name="s")

    @functools.partial(
        pl.kernel,
        mesh=mesh,
        compiler_params=pltpu.CompilerParams(needs_layout_passes=False),
        out_type=jax.ShapeDtypeStruct((B * _OUT,), jnp.float32),
        scratch_types=[
            pltpu.VMEM((_VOCAB_PAD * _OUT,), jnp.float32),
            pltpu.VMEM((_OUT * _LANES,), jnp.float32),
            pltpu.VMEM((chunk * L,), jnp.int32),
            pltpu.VMEM((chunk * _OUT,), jnp.float32),
        ],
    )
    def sc(ids_hbm, p_hbm, bias_hbm, out_hbm, p_v, bias_v, ids_v, out_v):
        wid = lax.axis_index("s") * _NC + lax.axis_index("c")
        base = wid * rows_per_tile
        pltpu.sync_copy(p_hbm, p_v)
        pltpu.sync_copy(bias_hbm, bias_v)
        lane = lax.iota(jnp.int32, 16)
        cols = [jnp.full((_LANES,), j, jnp.int32) for j in range(_OUT)]
        bias = [bias_v[pl.ds(j * _LANES, _LANES)] for j in range(_OUT)]

        for c in range(n_chunks):
            row0 = base + c * chunk
            pltpu.sync_copy(ids_hbm.at[pl.ds(row0 * L, chunk * L)], ids_v)
            for g in range(groups):
                rows = g * _LANES + lane
                row_base = rows * L

                def step(l, accs, row_base=row_base):
                    fl = jnp.full((_LANES,), l, jnp.int32)
                    idv = plsc.load_gather(ids_v, [row_base + fl])
                    id4 = idv * _OUT
                    return tuple(
                        acc + plsc.load_gather(p_v, [id4 + cols[j]])
                        for j, acc in enumerate(accs)
                    )

                accs = lax.fori_loop(0, L, step, tuple(bias))
                out_base = rows * _OUT
                for j in range(_OUT):
                    plsc.store_scatter(out_v, [out_base + cols[j]], accs[j])
            pltpu.sync_copy(out_v, out_hbm.at[pl.ds(row0 * _OUT, chunk * _OUT)])

    return sc


def kernel(input_ids, table, fc_w, fc_b):
    B, L = input_ids.shape
    ids = input_ids.astype(jnp.int32).reshape(B * L)
    tpad = (jnp.zeros((_VOCAB_PAD, _EMB), jnp.float32)
            .at[:table.shape[0]].set(table.astype(jnp.float32)))
    p = _project_table(tpad, fc_w.astype(jnp.float32), 1.0 / L).reshape(-1)
    bias = jnp.repeat(fc_b.astype(jnp.float32), _LANES)
    out_flat = _make_sc_kernel(B, L)(ids, p, bias)
    return out_flat.reshape(B, _OUT)
